# baseline (speedup 1.0000x reference)
import functools

import jax
import jax.numpy as jnp
from jax import lax
from jax.experimental import pallas as pl
from jax.experimental.pallas import tpu as pltpu
from jax.experimental.pallas import tpu_sc as plsc

N = 10000
E = 320000
D = 128
NC = 2
NS = 16
NW = NC * NS
FS = D // NW
CH = 2000
NCH = E // CH


def _lane_take(x, perm):
  dnums = lax.GatherDimensionNumbers(
      offset_dims=(), collapsed_slice_dims=(0,), start_index_map=(0,))
  return lax.gather(
      x, perm[:, None], dimension_numbers=dnums, slice_sizes=(1,),
      mode=lax.GatherScatterMode.PROMISE_IN_BOUNDS)


def _sc_segment_max(ht, src, dst, agg_init):
  mesh = plsc.VectorSubcoreMesh(
      core_axis_name="c", subcore_axis_name="s", num_cores=NC,
      num_subcores=NS)

  @functools.partial(
      pl.kernel,
      out_type=jax.ShapeDtypeStruct((NW, N * FS), jnp.float32),
      mesh=mesh,
      compiler_params=pltpu.CompilerParams(
          needs_layout_passes=False, use_tc_tiling_on_sc=False),
      scratch_types=[
          pltpu.VMEM((N * FS,), jnp.float32),
          pltpu.VMEM((N * FS,), jnp.float32),
          pltpu.VMEM((N * FS,), jnp.float32),
          pltpu.VMEM((CH,), jnp.int32),
          pltpu.VMEM((CH,), jnp.int32),
          pltpu.VMEM((CH,), jnp.int32),
          pltpu.VMEM((CH,), jnp.int32),
          pltpu.SemaphoreType.DMA,
          pltpu.SemaphoreType.DMA,
      ],
  )
  def k(ht_hbm, src_hbm, dst_hbm, init_hbm, out_hbm, h_v, agg_a, agg_b,
        src_a, dst_a, src_b, dst_b, sem_a, sem_b):
    wid = lax.axis_index("s") * NC + lax.axis_index("c")
    pltpu.sync_copy(ht_hbm.at[wid], h_v)
    pltpu.sync_copy(init_hbm, agg_a)
    pltpu.sync_copy(init_hbm, agg_b)

    iota = lax.iota(jnp.int32, 16)
    lane4 = iota & 3
    rep = iota >> 2

    def start(c, sv, dv, sem):
      pltpu.async_copy(src_hbm.at[pl.ds(c * CH, CH)], sv, sem)
      pltpu.async_copy(dst_hbm.at[pl.ds(c * CH, CH)], dv, sem)

    def wait(sv, dv, sem):
      pltpu.make_async_copy(src_hbm.at[pl.ds(0, CH)], sv, sem).wait()
      pltpu.make_async_copy(dst_hbm.at[pl.ds(0, CH)], dv, sem).wait()

    def one_group(g, sv, dv, agg_v):
      ridx = g * 4 + rep
      s = plsc.load_gather(sv, [ridx])
      d = plsc.load_gather(dv, [ridx])
      m = plsc.load_gather(h_v, [(s << 2) | lane4])
      val = m
      for r in (4, 8, 12):
        perm = (iota + r) & 15
        d2 = _lane_take(d, perm)
        v2 = _lane_take(m, perm)
        val = jnp.where(d2 == d, jnp.maximum(val, v2), val)
      aidx = (d << 2) | lane4
      cur = plsc.load_gather(agg_v, [aidx])
      plsc.store_scatter(agg_v, [aidx], jnp.maximum(cur, val))

    def process(sv, dv):
      def quad(t, _):
        one_group(t * 4 + 0, sv, dv, agg_a)
        one_group(t * 4 + 1, sv, dv, agg_b)
        one_group(t * 4 + 2, sv, dv, agg_a)
        one_group(t * 4 + 3, sv, dv, agg_b)
        return 0
      lax.fori_loop(0, CH // 16, quad, 0)

    start(0, src_a, dst_a, sem_a)

    def pair_body(i, _):
      c = 2 * i
      start(c + 1, src_b, dst_b, sem_b)
      wait(src_a, dst_a, sem_a)
      process(src_a, dst_a)

      @pl.when(c + 2 < NCH)
      def _():
        start(c + 2, src_a, dst_a, sem_a)

      wait(src_b, dst_b, sem_b)
      process(src_b, dst_b)
      return 0

    lax.fori_loop(0, NCH // 2, pair_body, 0)

    def merge(j, _):
      sl = pl.ds(j * 16, 16)
      agg_a[sl] = jnp.maximum(agg_a[sl], agg_b[sl])
      return 0

    lax.fori_loop(0, N * FS // 16, merge, 0)
    pltpu.sync_copy(agg_a, out_hbm.at[wid])

  return k(ht, src, dst, agg_init)


def _tc_post(h, agg, Wt, b2, g2, bt2):
  BLK = 1000

  def body(h_ref, a_ref, w_ref, b_ref, g_ref, bt_ref, o_ref):
    a = a_ref[...]
    a = jnp.where(a == -jnp.inf, 0.0, a)
    x = jnp.dot(h_ref[...] + a, w_ref[...],
                preferred_element_type=jnp.float32) + b_ref[...]
    x = jnp.maximum(x, 0.0)
    mu = jnp.mean(x, axis=-1, keepdims=True)
    xc = x - mu
    var = jnp.mean(xc * xc, axis=-1, keepdims=True)
    o_ref[...] = xc * lax.rsqrt(var + 1e-5) * g_ref[...] + bt_ref[...]

  return pl.pallas_call(
      body,
      grid=(N // BLK,),
      in_specs=[
          pl.BlockSpec((BLK, D), lambda i: (i, 0)),
          pl.BlockSpec((BLK, D), lambda i: (i, 0)),
          pl.BlockSpec((D, D), lambda i: (0, 0)),
          pl.BlockSpec((1, D), lambda i: (0, 0)),
          pl.BlockSpec((1, D), lambda i: (0, 0)),
          pl.BlockSpec((1, D), lambda i: (0, 0)),
      ],
      out_specs=pl.BlockSpec((BLK, D), lambda i: (i, 0)),
      out_shape=jax.ShapeDtypeStruct((N, D), jnp.float32),
  )(h, agg, Wt, b2, g2, bt2)


@jax.jit
def kernel(h, edge_index, W, b, ln_gamma, ln_beta):
  ht = h.reshape(N, NW, FS).transpose(1, 0, 2).reshape(NW, N * FS)
  src = edge_index[0]
  dst = edge_index[1]
  agg_init = jnp.full((N * FS,), -jnp.inf, jnp.float32)
  aggT = _sc_segment_max(ht, src, dst, agg_init)
  agg = aggT.reshape(NW, N, FS).transpose(1, 0, 2).reshape(N, D)
  return _tc_post(h, agg, W.T, b.reshape(1, D), ln_gamma.reshape(1, D),
                  ln_beta.reshape(1, D))


# speedup vs baseline: 2.0331x; 2.0331x over previous
import functools

import jax
import jax.numpy as jnp
from jax import lax
from jax.experimental import pallas as pl
from jax.experimental.pallas import tpu as pltpu
from jax.experimental.pallas import tpu_sc as plsc

N = 10000
E = 320000
D = 128
NC = 2
NS = 16
NW = NC * NS
FS = D // NW
CH = 4000
NCH = E // CH
G = 8


def _lane_take(x, perm):
  dnums = lax.GatherDimensionNumbers(
      offset_dims=(), collapsed_slice_dims=(0,), start_index_map=(0,))
  return lax.gather(
      x, perm[:, None], dimension_numbers=dnums, slice_sizes=(1,),
      mode=lax.GatherScatterMode.PROMISE_IN_BOUNDS)


def _sc_segment_max(ht, src, dst, agg_init):
  mesh = plsc.VectorSubcoreMesh(
      core_axis_name="c", subcore_axis_name="s", num_cores=NC,
      num_subcores=NS)

  @functools.partial(
      pl.kernel,
      out_type=jax.ShapeDtypeStruct((NW, N * FS), jnp.float32),
      mesh=mesh,
      compiler_params=pltpu.CompilerParams(
          needs_layout_passes=False, use_tc_tiling_on_sc=False),
      scratch_types=[
          pltpu.VMEM((N * FS,), jnp.float32),
          pltpu.VMEM((N * FS,), jnp.float32),
          pltpu.VMEM((CH,), jnp.int32),
          pltpu.VMEM((CH,), jnp.int32),
          pltpu.VMEM((CH,), jnp.int32),
          pltpu.VMEM((CH,), jnp.int32),
          pltpu.SemaphoreType.DMA,
          pltpu.SemaphoreType.DMA,
      ],
  )
  def k(ht_hbm, src_hbm, dst_hbm, init_hbm, out_hbm, h_v, agg_v,
        src_a, dst_a, src_b, dst_b, sem_a, sem_b):
    wid = lax.axis_index("s") * NC + lax.axis_index("c")
    pltpu.sync_copy(ht_hbm.at[wid], h_v)
    pltpu.sync_copy(init_hbm, agg_v)

    iota = lax.iota(jnp.int32, 16)
    lane4 = iota & 3
    rep = iota >> 2

    def start(c, sv, dv, sem):
      pltpu.async_copy(src_hbm.at[pl.ds(c * CH, CH)], sv, sem)
      pltpu.async_copy(dst_hbm.at[pl.ds(c * CH, CH)], dv, sem)

    def wait(sv, dv, sem):
      pltpu.make_async_copy(src_hbm.at[pl.ds(0, CH)], sv, sem).wait()
      pltpu.make_async_copy(dst_hbm.at[pl.ds(0, CH)], dv, sem).wait()

    def process(sv, dv):
      def batch(t, base):
        ms, ds, aidxs, news = [], [], [], []
        for g in range(G):
          idx = base + 4 * g
          s = plsc.load_gather(sv, [idx])
          d = plsc.load_gather(dv, [idx])
          m = plsc.load_gather(h_v, [(s << 2) | lane4])
          aidx = (d << 2) | lane4
          cur = plsc.load_gather(agg_v, [aidx])
          ms.append(m)
          ds.append(d)
          aidxs.append(aidx)
          news.append(jnp.maximum(cur, m))
        for g in range(G):
          plsc.store_scatter(agg_v, [aidxs[g]], news[g])
        viol = None
        for g in range(G):
          r = plsc.load_gather(agg_v, [aidxs[g]])
          v = r < ms[g]
          viol = v if viol is None else jnp.logical_or(viol, v)

        @pl.when(jnp.any(viol))
        def _():
          for g in range(G):
            m, d, aidx = ms[g], ds[g], aidxs[g]
            cs = [m]
            for r_ in (4, 8, 12):
              perm = (iota + r_) & 15
              d2 = _lane_take(d, perm)
              v2 = _lane_take(m, perm)
              cs.append(jnp.where(d2 == d, v2, m))
            val = jnp.maximum(jnp.maximum(cs[0], cs[1]),
                              jnp.maximum(cs[2], cs[3]))
            cur = plsc.load_gather(agg_v, [aidx])
            plsc.store_scatter(agg_v, [aidx], jnp.maximum(cur, val))

        return base + 4 * G

      lax.fori_loop(0, CH // (4 * G), batch, rep)

    start(0, src_a, dst_a, sem_a)

    def pair_body(i, _):
      c = 2 * i
      start(c + 1, src_b, dst_b, sem_b)
      wait(src_a, dst_a, sem_a)
      process(src_a, dst_a)

      @pl.when(c + 2 < NCH)
      def _():
        start(c + 2, src_a, dst_a, sem_a)

      wait(src_b, dst_b, sem_b)
      process(src_b, dst_b)
      return 0

    lax.fori_loop(0, NCH // 2, pair_body, 0)
    pltpu.sync_copy(agg_v, out_hbm.at[wid])

  return k(ht, src, dst, agg_init)


def _tc_post(h, agg, Wt, b2, g2, bt2):
  BLK = 1000

  def body(h_ref, a_ref, w_ref, b_ref, g_ref, bt_ref, o_ref):
    a = a_ref[...]
    a = jnp.where(a == -jnp.inf, 0.0, a)
    x = jnp.dot(h_ref[...] + a, w_ref[...],
                preferred_element_type=jnp.float32) + b_ref[...]
    x = jnp.maximum(x, 0.0)
    mu = jnp.mean(x, axis=-1, keepdims=True)
    xc = x - mu
    var = jnp.mean(xc * xc, axis=-1, keepdims=True)
    o_ref[...] = xc * lax.rsqrt(var + 1e-5) * g_ref[...] + bt_ref[...]

  return pl.pallas_call(
      body,
      grid=(N // BLK,),
      in_specs=[
          pl.BlockSpec((BLK, D), lambda i: (i, 0)),
          pl.BlockSpec((BLK, D), lambda i: (i, 0)),
          pl.BlockSpec((D, D), lambda i: (0, 0)),
          pl.BlockSpec((1, D), lambda i: (0, 0)),
          pl.BlockSpec((1, D), lambda i: (0, 0)),
          pl.BlockSpec((1, D), lambda i: (0, 0)),
      ],
      out_specs=pl.BlockSpec((BLK, D), lambda i: (i, 0)),
      out_shape=jax.ShapeDtypeStruct((N, D), jnp.float32),
  )(h, agg, Wt, b2, g2, bt2)


@jax.jit
def kernel(h, edge_index, W, b, ln_gamma, ln_beta):
  ht = h.reshape(N, NW, FS).transpose(1, 0, 2).reshape(NW, N * FS)
  src = edge_index[0]
  dst = edge_index[1]
  agg_init = jnp.full((N * FS,), -jnp.inf, jnp.float32)
  aggT = _sc_segment_max(ht, src, dst, agg_init)
  agg = aggT.reshape(NW, N, FS).transpose(1, 0, 2).reshape(N, D)
  return _tc_post(h, agg, W.T, b.reshape(1, D), ln_gamma.reshape(1, D),
                  ln_beta.reshape(1, D))


# speedup vs baseline: 3.1261x; 1.5376x over previous
import functools

import jax
import jax.numpy as jnp
from jax import lax
from jax.experimental import pallas as pl
from jax.experimental.pallas import tpu as pltpu
from jax.experimental.pallas import tpu_sc as plsc

N = 10000
E = 320000
D = 128
NC = 2
NS = 16
NW = NC * NS
FS = D // NW
CH = 4000
NCH = E // CH
G = 8


def _lane_take(x, perm):
  dnums = lax.GatherDimensionNumbers(
      offset_dims=(), collapsed_slice_dims=(0,), start_index_map=(0,))
  return lax.gather(
      x, perm[:, None], dimension_numbers=dnums, slice_sizes=(1,),
      mode=lax.GatherScatterMode.PROMISE_IN_BOUNDS)


def _sc_segment_max(ht, src, dst, agg_init):
  mesh = plsc.VectorSubcoreMesh(
      core_axis_name="c", subcore_axis_name="s", num_cores=NC,
      num_subcores=NS)

  @functools.partial(
      pl.kernel,
      out_type=jax.ShapeDtypeStruct((NW, N * FS), jnp.float32),
      mesh=mesh,
      compiler_params=pltpu.CompilerParams(
          needs_layout_passes=False, use_tc_tiling_on_sc=False),
      scratch_types=[
          pltpu.VMEM((N * FS,), jnp.float32),
          pltpu.VMEM((N * FS,), jnp.float32),
          pltpu.VMEM((CH,), jnp.int32),
          pltpu.VMEM((CH,), jnp.int32),
          pltpu.VMEM((CH,), jnp.int32),
          pltpu.VMEM((CH,), jnp.int32),
          pltpu.SemaphoreType.DMA,
          pltpu.SemaphoreType.DMA,
      ],
  )
  def k(ht_hbm, src_hbm, dst_hbm, init_hbm, out_hbm, h_v, agg_v,
        src_a, dst_a, src_b, dst_b, sem_a, sem_b):
    wid = lax.axis_index("s") * NC + lax.axis_index("c")
    pltpu.sync_copy(ht_hbm.at[wid], h_v)
    pltpu.sync_copy(init_hbm, agg_v)

    iota = lax.iota(jnp.int32, 16)
    lane4 = iota & 3
    rep = iota >> 2
    reps = [rep, rep + 4, rep + 8, rep + 12]
    zeros = jnp.zeros((16,), jnp.float32)

    def start(c, sv, dv, sem):
      pltpu.async_copy(src_hbm.at[pl.ds(c * CH, CH)], sv, sem)
      pltpu.async_copy(dst_hbm.at[pl.ds(c * CH, CH)], dv, sem)

    def wait(sv, dv, sem):
      pltpu.make_async_copy(src_hbm.at[pl.ds(0, CH)], sv, sem).wait()
      pltpu.make_async_copy(dst_hbm.at[pl.ds(0, CH)], dv, sem).wait()

    def exact_batch(off, sv, dv):
      for g in range(G):
        idx = (off + 4 * g) + rep
        s = plsc.load_gather(sv, [idx])
        d = plsc.load_gather(dv, [idx])
        m = plsc.load_gather(h_v, [(s << 2) | lane4])
        cs = [m]
        for r_ in (4, 8, 12):
          perm = (iota + r_) & 15
          d2 = _lane_take(d, perm)
          v2 = _lane_take(m, perm)
          cs.append(jnp.where(d2 == d, v2, m))
        val = jnp.maximum(jnp.maximum(cs[0], cs[1]),
                          jnp.maximum(cs[2], cs[3]))
        aidx = (d << 2) | lane4
        cur = plsc.load_gather(agg_v, [aidx])
        plsc.store_scatter(agg_v, [aidx], jnp.maximum(cur, val))

    def process(sv, dv):
      def batch(t, carry):
        pviolf, poff = carry
        off = t * (4 * G)
        prev_bad = jnp.max(pviolf) > 0.0

        sraw = [sv[pl.ds(off, 16)] << 2, sv[pl.ds(off + 16, 16)] << 2]
        draw = [dv[pl.ds(off, 16)] << 2, dv[pl.ds(off + 16, 16)] << 2]

        ms, aidxs, news = [], [], []
        for g in range(G):
          s4 = _lane_take(sraw[g // 4], reps[g % 4])
          d4 = _lane_take(draw[g // 4], reps[g % 4])
          m = plsc.load_gather(h_v, [s4 | lane4])
          aidx = d4 | lane4
          cur = plsc.load_gather(agg_v, [aidx])
          ms.append(m)
          aidxs.append(aidx)
          news.append(jnp.maximum(cur, m))
        for g in range(G):
          plsc.store_scatter(agg_v, [aidxs[g]], news[g])
        viol = None
        for g in range(G):
          r = plsc.load_gather(agg_v, [aidxs[g]])
          v = r < ms[g]
          viol = v if viol is None else jnp.logical_or(viol, v)
        violf = jnp.where(viol, 1.0, 0.0)

        @pl.when(prev_bad)
        def _():
          exact_batch(poff, sv, dv)

        return (violf, off)

      violf_f, poff_f = lax.fori_loop(
          0, CH // (4 * G), batch, (zeros, 0))

      @pl.when(jnp.max(violf_f) > 0.0)
      def _():
        exact_batch(poff_f, sv, dv)

    start(0, src_a, dst_a, sem_a)

    def pair_body(i, _):
      c = 2 * i
      start(c + 1, src_b, dst_b, sem_b)
      wait(src_a, dst_a, sem_a)
      process(src_a, dst_a)

      @pl.when(c + 2 < NCH)
      def _():
        start(c + 2, src_a, dst_a, sem_a)

      wait(src_b, dst_b, sem_b)
      process(src_b, dst_b)
      return 0

    lax.fori_loop(0, NCH // 2, pair_body, 0)
    pltpu.sync_copy(agg_v, out_hbm.at[wid])

  return k(ht, src, dst, agg_init)


def _tc_post(h, agg, Wt, b2, g2, bt2):
  BLK = 1000

  def body(h_ref, a_ref, w_ref, b_ref, g_ref, bt_ref, o_ref):
    a = a_ref[...]
    a = jnp.where(a == -jnp.inf, 0.0, a)
    x = jnp.dot(h_ref[...] + a, w_ref[...],
                preferred_element_type=jnp.float32) + b_ref[...]
    x = jnp.maximum(x, 0.0)
    mu = jnp.mean(x, axis=-1, keepdims=True)
    xc = x - mu
    var = jnp.mean(xc * xc, axis=-1, keepdims=True)
    o_ref[...] = xc * lax.rsqrt(var + 1e-5) * g_ref[...] + bt_ref[...]

  return pl.pallas_call(
      body,
      grid=(N // BLK,),
      in_specs=[
          pl.BlockSpec((BLK, D), lambda i: (i, 0)),
          pl.BlockSpec((BLK, D), lambda i: (i, 0)),
          pl.BlockSpec((D, D), lambda i: (0, 0)),
          pl.BlockSpec((1, D), lambda i: (0, 0)),
          pl.BlockSpec((1, D), lambda i: (0, 0)),
          pl.BlockSpec((1, D), lambda i: (0, 0)),
      ],
      out_specs=pl.BlockSpec((BLK, D), lambda i: (i, 0)),
      out_shape=jax.ShapeDtypeStruct((N, D), jnp.float32),
  )(h, agg, Wt, b2, g2, bt2)


@jax.jit
def kernel(h, edge_index, W, b, ln_gamma, ln_beta):
  ht = h.reshape(N, NW, FS).transpose(1, 0, 2).reshape(NW, N * FS)
  src = edge_index[0]
  dst = edge_index[1]
  agg_init = jnp.full((N * FS,), -jnp.inf, jnp.float32)
  aggT = _sc_segment_max(ht, src, dst, agg_init)
  agg = aggT.reshape(NW, N, FS).transpose(1, 0, 2).reshape(N, D)
  return _tc_post(h, agg, W.T, b.reshape(1, D), ln_gamma.reshape(1, D),
                  ln_beta.reshape(1, D))
